# baseline (speedup 1.0000x reference)
import functools

import jax
import jax.numpy as jnp
from jax import lax
from jax.experimental import pallas as pl
from jax.experimental.pallas import tpu as pltpu
from jax.experimental.pallas import tpu_sc as plsc
from jax._src.pallas import mpmd

_I = 13
_LB = -500000.0
_N_STATES = 1000001

_B = 16384
_NC = 1
_NS = 16
_NW = _NC * _NS
_BPW = _B // _NW
_CHUNK = 128
_NCHUNK = _BPW // _CHUNK
_LANES = 16
_MAGIC = 8388608.0

_scalar_mesh = plsc.ScalarSubcoreMesh(axis_name="c", num_cores=1)
_vector_mesh = plsc.VectorSubcoreMesh(core_axis_name="c", subcore_axis_name="s", num_cores=1)


def _scs_body(xi_hbm, table_hbm, out_hbm, xi_v, idx_v, val_v, tab_s, sem, rdy,
              ssem):
    pltpu.sync_copy(table_hbm, tab_s)
    for i in range(_NS):
        pl.semaphore_signal(rdy, 1, device_id={"s": i})


def _tec_body(xi_hbm, table_hbm, out_hbm, xi_v, idx_v, val_v, tab_s, sem, rdy,
              ssem):
    cid = lax.axis_index("c")
    sid = lax.axis_index("s")
    wid = sid * _NC + cid
    base = wid * _BPW

    pltpu.sync_copy(xi_hbm.at[pl.ds(base, _BPW)], xi_v)

    groups_per_chunk = _CHUNK // _LANES
    for j in range(_BPW // _LANES):
        xi = xi_v[pl.ds(j * _LANES, _LANES)]
        v = xi - _LB
        v = jnp.minimum(jnp.maximum(v, 0.0), float(_N_STATES - 1))
        v = (v + _MAGIC) - _MAGIC
        idx = v.astype(jnp.int32)
        idx_v[j // groups_per_chunk,
              pl.ds((j % groups_per_chunk) * _LANES, _LANES)] = idx

    pl.semaphore_wait(rdy, 1)

    gathers = [
        pltpu.async_copy(
            tab_s.at[idx_v.at[c]],
            val_v.at[pl.ds(c * _CHUNK, _CHUNK)],
            sem,
        )
        for c in range(_NCHUNK)
    ]
    for cp in gathers:
        cp.wait()

    pltpu.sync_copy(val_v, out_hbm.at[pl.ds(base, _BPW)])


_table_gather = mpmd.mpmd_map(
    [(_scalar_mesh, _scs_body), (_vector_mesh, _tec_body)],
    jax.ShapeDtypeStruct((_B,), jnp.float32),
    scratch_types=[
        pltpu.VMEM((_BPW,), jnp.float32) @ _vector_mesh,
        pltpu.VMEM((_NCHUNK, _CHUNK), jnp.int32) @ _vector_mesh,
        pltpu.VMEM((_BPW,), jnp.float32) @ _vector_mesh,
        pltpu.VMEM_SHARED((_N_STATES,), jnp.float32),
        pltpu.SemaphoreType.DMA @ _vector_mesh,
        pltpu.SemaphoreType.REGULAR @ _vector_mesh,
        pltpu.SemaphoreType.DMA @ _scalar_mesh,
    ],
)


def kernel(x, table):
    return _table_gather(lax.slice(x, (0, _I), (_B, _I + 1)).reshape(_B), table)[:, None]


# speedup vs baseline: 1.0147x; 1.0147x over previous
import functools

import jax
import jax.numpy as jnp
from jax import lax
from jax.experimental import pallas as pl
from jax.experimental.pallas import tpu as pltpu
from jax.experimental.pallas import tpu_sc as plsc
from jax._src.pallas import mpmd

_I = 13
_LB = -500000.0
_N_STATES = 1000001

_B = 16384
_NC = 2
_NS = 16
_NW = _NC * _NS
_BPW = _B // _NW
_CHUNK = 128
_NCHUNK = _BPW // _CHUNK
_LANES = 16
_MAGIC = 8388608.0

_scalar_mesh = plsc.ScalarSubcoreMesh(axis_name="c", num_cores=_NC)
_vector_mesh = plsc.VectorSubcoreMesh(core_axis_name="c", subcore_axis_name="s")


def _scs_body(xi_hbm, table_hbm, out_hbm, xi_v, idx_v, val_v, tab_s,
              g0, g1, g2, g3, osem, rdy):
    pltpu.sync_copy(table_hbm, tab_s)
    for i in range(_NS):
        pl.semaphore_signal(rdy, 1, device_id={"s": i})


def _tec_body(xi_hbm, table_hbm, out_hbm, xi_v, idx_v, val_v, tab_s,
              g0, g1, g2, g3, osem, rdy):
    cid = lax.axis_index("c")
    sid = lax.axis_index("s")
    wid = sid * _NC + cid
    base = wid * _BPW

    pltpu.sync_copy(xi_hbm.at[pl.ds(base, _BPW)], xi_v)

    groups_per_chunk = _CHUNK // _LANES
    for j in range(_BPW // _LANES):
        xi = xi_v[pl.ds(j * _LANES, _LANES)]
        v = xi - _LB
        v = jnp.minimum(jnp.maximum(v, 0.0), float(_N_STATES - 1))
        v = (v + _MAGIC) - _MAGIC
        idx = v.astype(jnp.int32)
        idx_v[j // groups_per_chunk,
              pl.ds((j % groups_per_chunk) * _LANES, _LANES)] = idx

    pl.semaphore_wait(rdy, 1)

    gsems = (g0, g1, g2, g3)
    gathers = [
        pltpu.async_copy(
            tab_s.at[idx_v.at[c]],
            val_v.at[pl.ds(c * _CHUNK, _CHUNK)],
            gsems[c],
        )
        for c in range(_NCHUNK)
    ]
    outs = []
    for c in range(_NCHUNK):
        gathers[c].wait()
        outs.append(pltpu.async_copy(
            val_v.at[pl.ds(c * _CHUNK, _CHUNK)],
            out_hbm.at[pl.ds(base + c * _CHUNK, _CHUNK)],
            osem))
    for cp in outs:
        cp.wait()


_table_gather = mpmd.mpmd_map(
    [(_scalar_mesh, _scs_body), (_vector_mesh, _tec_body)],
    jax.ShapeDtypeStruct((_B,), jnp.float32),
    scratch_types=[
        pltpu.VMEM((_BPW,), jnp.float32) @ _vector_mesh,
        pltpu.VMEM((_NCHUNK, _CHUNK), jnp.int32) @ _vector_mesh,
        pltpu.VMEM((_BPW,), jnp.float32) @ _vector_mesh,
        pltpu.VMEM_SHARED((_N_STATES,), jnp.float32),
        pltpu.SemaphoreType.DMA @ _vector_mesh,
        pltpu.SemaphoreType.DMA @ _vector_mesh,
        pltpu.SemaphoreType.DMA @ _vector_mesh,
        pltpu.SemaphoreType.DMA @ _vector_mesh,
        pltpu.SemaphoreType.DMA @ _vector_mesh,
        pltpu.SemaphoreType.REGULAR @ _vector_mesh,
    ],
)


def kernel(x, table):
    return _table_gather(lax.slice(x, (0, _I), (_B, _I + 1)).reshape(_B), table)[:, None]
